# baseline (speedup 1.0000x reference)
import jax
import jax.numpy as jnp
from jax import lax
from jax.experimental import pallas as pl
from jax.experimental.pallas import tpu as pltpu
from jax.experimental.pallas import tpu_sc as plsc

N = 100000
E = 3200000
C = 10
PW = 16
NC, NS = 2, 16
NW = NC * NS

GSZ = 128
NG = E // GSZ
CG = 4
CE = CG * GSZ
NP = NG // (2 * CG)
BASE_P = NP // NW
EXTRA_P = NP % NW
QMAX = ((BASE_P + 1) * 2) // 4

ZBLK = (N // 8) // NS
ZEXTRA = (N // 8) % NS

GRP = N // 16
BASE_R = GRP // NW
EXTRA_R = GRP % NW
CGR = 40
FULL2 = BASE_R // CGR
TAIL2 = BASE_R - FULL2 * CGR

_mesh = plsc.VectorSubcoreMesh(
    core_axis_name="c", subcore_axis_name="s", num_cores=NC, num_subcores=NS
)
_params = pltpu.CompilerParams(
    needs_layout_passes=False,
    use_tc_tiling_on_sc=False,
)


def _scatter_body(
    pe_hbm, w_hbm, src_hbm, part_hbm,
    idx0, idx1, w0, w1, pe0, pe1, val0, val1,
    table,
    sin0, sin1, sidx0, sidx1, sout0, sout1,
):
    idx_v = [idx0, idx1]
    w_v = [w0, w1]
    pe_v = [pe0, pe1]
    val_v = [val0, val1]
    sin = [sin0, sin1]
    sidx = [sidx0, sidx1]
    sout = [sout0, sout1]

    cid = lax.axis_index("c")
    sid = lax.axis_index("s")
    wid = sid * NC + cid
    iota = lax.iota(jnp.int32, 16)
    cols = [jnp.full((16,), j, jnp.int32) for j in range(C + 1)]
    zeros16 = jnp.zeros((16,), jnp.float32)

    def _zero_row(i, carry):
        val0[i, :] = zeros16
        val1[i, :] = zeros16
        return carry

    lax.fori_loop(0, CE, _zero_row, 0)

    r0 = (ZBLK * sid + jnp.minimum(sid, ZEXTRA)) * 8
    zrows = ZBLK * 8
    zfull = zrows // CE
    zrem = zrows - zfull * CE
    for t in range(zfull):
        pltpu.sync_copy(val0, table.at[pl.ds(r0 + t * CE, CE)])
    pltpu.sync_copy(
        val0.at[pl.ds(0, zrem)], table.at[pl.ds(r0 + zfull * CE, zrem)]
    )

    @pl.when(sid < ZEXTRA)
    def _():
        pltpu.sync_copy(val0.at[pl.ds(0, 8)], table.at[pl.ds(r0 + zrows, 8)])

    plsc.subcore_barrier()

    my_p = BASE_P + jnp.where(wid < EXTRA_P, 1, 0)
    my_n = 2 * my_p
    ch0 = (BASE_P * wid + jnp.minimum(wid, EXTRA_P)) * 2

    def start_in(cc, b):
        eb = (ch0 + cc) * CE
        pltpu.async_copy(w_hbm.at[pl.ds(eb, CE)], w_v[b], sin[b])
        pltpu.async_copy(pe_hbm.at[pl.ds(eb * C, CE * C)], pe_v[b], sin[b])

    def drain_in(cc, b):
        eb = (ch0 + cc) * CE
        pltpu.make_async_copy(w_hbm.at[pl.ds(eb, CE)], w_v[b], sin[b]).wait()
        pltpu.make_async_copy(
            pe_hbm.at[pl.ds(eb * C, CE * C)], pe_v[b], sin[b]
        ).wait()

    def start_idx(p, pb):
        gp = ch0 * CG + p * 2 * CG
        pltpu.async_copy(src_hbm.at[pl.ds(gp, 2 * CG)], idx_v[pb], sidx[pb])

    def drain_idx(p, pb):
        gp = ch0 * CG + p * 2 * CG
        pltpu.make_async_copy(
            src_hbm.at[pl.ds(gp, 2 * CG)], idx_v[pb], sidx[pb]
        ).wait()

    def drain_out(b):
        pltpu.make_async_copy(
            part_hbm.at[0, pl.ds(0, CE)], val_v[b], sout[b]
        ).wait()

    def compute(b):
        def body(k, carry):
            row_idx, pidx = carry
            wv = w_v[b][pl.ds(k * 16, 16)]
            for j in range(C):
                g = plsc.load_gather(pe_v[b], [pidx + j])
                plsc.store_scatter(val_v[b], [row_idx, cols[j]], g * wv)
            plsc.store_scatter(val_v[b], [row_idx, cols[C]], wv)
            return (row_idx + 16, pidx + 16 * C)

        lax.fori_loop(0, CE // 16, body, (iota, iota * C))

    def scatter(b, pb, half):
        for j in range(CG):
            pltpu.async_copy(
                val_v[b].at[pl.ds(j * GSZ, GSZ)],
                table.at[idx_v[pb].at[half * CG + j]],
                sout[b],
                add=True,
            )

    start_idx(0, 0)
    start_in(0, 0)

    def qbody(q, carry):
        for k in range(4):
            cc = 4 * q + k
            b = k & 1
            p = 2 * q + (k >> 1)
            pb = (k >> 1) & 1

            @pl.when(cc < my_n)
            def _(k=k, cc=cc, b=b, p=p, pb=pb):
                if k % 2 == 0:
                    drain_idx(p, pb)

                @pl.when(cc >= 2)
                def _():
                    drain_out(b)

                if k % 2 == 1:
                    @pl.when(cc + 1 < my_n)
                    def _():
                        start_idx(p + 1, 1 - pb)

                drain_in(cc, b)

                @pl.when(cc + 1 < my_n)
                def _():
                    start_in(cc + 1, 1 - b)

                compute(b)
                scatter(b, pb, k & 1)
        return carry

    lax.fori_loop(0, QMAX, qbody, 0)

    drain_out(0)
    drain_out(1)

    plsc.subcore_barrier()

    def dump(rbase, nrows):
        pltpu.sync_copy(
            table.at[pl.ds(rbase, nrows)], val0.at[pl.ds(0, nrows)]
        )
        pltpu.sync_copy(
            val0.at[pl.ds(0, nrows)], part_hbm.at[cid, pl.ds(rbase, nrows)]
        )

    for t in range(zfull):
        dump(r0 + t * CE, CE)
    dump(r0 + zfull * CE, zrem)

    @pl.when(sid < ZEXTRA)
    def _():
        dump(r0 + zrows, 8)


_scatter = pl.kernel(
    _scatter_body,
    out_type=jax.ShapeDtypeStruct((NC, N, PW), jnp.float32),
    mesh=_mesh,
    compiler_params=_params,
    scratch_types=[
        pltpu.VMEM((2 * CG, GSZ), jnp.int32),
        pltpu.VMEM((2 * CG, GSZ), jnp.int32),
        pltpu.VMEM((CE,), jnp.float32),
        pltpu.VMEM((CE,), jnp.float32),
        pltpu.VMEM((CE * C,), jnp.float32),
        pltpu.VMEM((CE * C,), jnp.float32),
        pltpu.VMEM((CE, PW), jnp.float32),
        pltpu.VMEM((CE, PW), jnp.float32),
        pltpu.VMEM_SHARED((N, PW), jnp.float32),
        pltpu.SemaphoreType.DMA,
        pltpu.SemaphoreType.DMA,
        pltpu.SemaphoreType.DMA,
        pltpu.SemaphoreType.DMA,
        pltpu.SemaphoreType.DMA,
        pltpu.SemaphoreType.DMA,
    ],
)


def _finish_body(part_hbm, out_hbm, p0_v, p1_v, o_v):
    cid = lax.axis_index("c")
    sid = lax.axis_index("s")
    wid = sid * NC + cid
    iota = lax.iota(jnp.int32, 16)
    cols = [jnp.full((16,), j, jnp.int32) for j in range(C + 1)]

    gr0 = BASE_R * wid + jnp.minimum(wid, EXTRA_R)

    def process(grb, ng):
        rb = grb * 16
        nr = ng * 16
        pltpu.sync_copy(part_hbm.at[0, pl.ds(rb, nr)], p0_v.at[pl.ds(0, nr)])
        pltpu.sync_copy(part_hbm.at[1, pl.ds(rb, nr)], p1_v.at[pl.ds(0, nr)])

        def gbody(g, row_idx):
            den = jnp.maximum(
                plsc.load_gather(p0_v, [row_idx, cols[C]])
                + plsc.load_gather(p1_v, [row_idx, cols[C]]),
                1e-12,
            )
            for j in range(C):
                s = plsc.load_gather(p0_v, [row_idx, cols[j]]) + plsc.load_gather(
                    p1_v, [row_idx, cols[j]]
                )
                plsc.store_scatter(o_v, [row_idx, cols[j]], s / den)
            return row_idx + 16

        lax.fori_loop(0, ng, gbody, iota)
        pltpu.sync_copy(o_v.at[pl.ds(0, nr)], out_hbm.at[pl.ds(rb, nr)])

    def chunk(cc, carry):
        process(gr0 + cc * CGR, CGR)
        return carry

    lax.fori_loop(0, FULL2, chunk, 0)
    process(gr0 + FULL2 * CGR, TAIL2)

    @pl.when(wid < EXTRA_R)
    def _():
        process(gr0 + BASE_R, 1)


_finish = pl.kernel(
    _finish_body,
    out_type=jax.ShapeDtypeStruct((N, PW), jnp.float32),
    mesh=_mesh,
    compiler_params=_params,
    scratch_types=[
        pltpu.VMEM((CGR * 16, PW), jnp.float32),
        pltpu.VMEM((CGR * 16, PW), jnp.float32),
        pltpu.VMEM((CGR * 16, PW), jnp.float32),
    ],
)


def kernel(poss_edge, weights, edges):
    pe_flat = poss_edge.reshape(E * C)
    src2d = edges[:, 0].reshape(NG, GSZ)
    pe_flat, src2d, weights = lax.optimization_barrier(
        (pe_flat, src2d, weights)
    )
    valt = jnp.concatenate(
        [
            poss_edge.T * weights[None, :],
            weights[None, :],
            jnp.zeros((5, E), jnp.float32),
        ],
        axis=0,
    )
    (valt,) = lax.optimization_barrier((valt,))
    part = _scatter(pe_flat, weights, src2d)
    out = _finish(part)
    return out[:, :C] + valt[0, 0] * 1e-30, poss_edge


# speedup vs baseline: 2.4172x; 2.4172x over previous
import jax
import jax.numpy as jnp
from jax import lax
from jax.experimental import pallas as pl
from jax.experimental.pallas import tpu as pltpu
from jax.experimental.pallas import tpu_sc as plsc

N = 100000
E = 3200000
C = 10
PW = 16
NC, NS = 2, 16
NW = NC * NS

GSZ = 128
NG = E // GSZ
CG = 4
CE = CG * GSZ
NP = NG // (2 * CG)
BASE_P = NP // NW
EXTRA_P = NP % NW
QMAX = ((BASE_P + 1) * 2) // 4

ZBLK = (N // 8) // NS
ZEXTRA = (N // 8) % NS

GRP = N // 16
BASE_R = GRP // NW
EXTRA_R = GRP % NW
CGR = 40
FULL2 = BASE_R // CGR
TAIL2 = BASE_R - FULL2 * CGR

_mesh = plsc.VectorSubcoreMesh(
    core_axis_name="c", subcore_axis_name="s", num_cores=NC, num_subcores=NS
)
_params = pltpu.CompilerParams(
    needs_layout_passes=False,
    use_tc_tiling_on_sc=False,
)


def _scatter_body(
    vt_hbm, src_hbm, part_hbm,
    idx0, idx1, vt0, vt1, val_v,
    table,
    sidx0, sidx1, sin0, sin1, sout,
):
    idx_v = [idx0, idx1]
    vt_v = [vt0, vt1]
    sidx = [sidx0, sidx1]
    sin = [sin0, sin1]

    cid = lax.axis_index("c")
    sid = lax.axis_index("s")
    wid = sid * NC + cid
    iota = lax.iota(jnp.int32, 16)
    cols = [jnp.full((16,), j, jnp.int32) for j in range(C + 1)]
    zeros16 = jnp.zeros((16,), jnp.float32)

    def _zero_row(i, carry):
        val_v[i, :] = zeros16
        return carry

    lax.fori_loop(0, CE, _zero_row, 0)

    r0 = (ZBLK * sid + jnp.minimum(sid, ZEXTRA)) * 8
    zrows = ZBLK * 8
    zfull = zrows // CE
    zrem = zrows - zfull * CE
    for t in range(zfull):
        pltpu.sync_copy(val_v, table.at[pl.ds(r0 + t * CE, CE)])
    pltpu.sync_copy(
        val_v.at[pl.ds(0, zrem)], table.at[pl.ds(r0 + zfull * CE, zrem)]
    )

    @pl.when(sid < ZEXTRA)
    def _():
        pltpu.sync_copy(val_v.at[pl.ds(0, 8)], table.at[pl.ds(r0 + zrows, 8)])

    plsc.subcore_barrier()

    my_p = BASE_P + jnp.where(wid < EXTRA_P, 1, 0)
    my_n = 2 * my_p
    ch0 = (BASE_P * wid + jnp.minimum(wid, EXTRA_P)) * 2

    def start_vt(cc, b):
        eb = (ch0 + cc) * CE
        pltpu.async_copy(vt_hbm.at[:, pl.ds(eb, CE)], vt_v[b], sin[b])

    def drain_vt(cc, b):
        eb = (ch0 + cc) * CE
        pltpu.make_async_copy(
            vt_hbm.at[:, pl.ds(eb, CE)], vt_v[b], sin[b]
        ).wait()

    def start_idx(p, pb):
        gp = ch0 * CG + p * 2 * CG
        pltpu.async_copy(src_hbm.at[pl.ds(gp, 2 * CG)], idx_v[pb], sidx[pb])

    def drain_idx(p, pb):
        gp = ch0 * CG + p * 2 * CG
        pltpu.make_async_copy(
            src_hbm.at[pl.ds(gp, 2 * CG)], idx_v[pb], sidx[pb]
        ).wait()

    def drain_out():
        pltpu.make_async_copy(
            part_hbm.at[0, pl.ds(0, CE)], val_v, sout
        ).wait()

    def compute(b):
        def body(k, row_idx):
            for j in range(C + 1):
                v = vt_v[b][j, pl.ds(k * 16, 16)]
                plsc.store_scatter(val_v, [row_idx, cols[j]], v)
            return row_idx + 16

        lax.fori_loop(0, CE // 16, body, iota)

    def scatter(pb, half):
        for j in range(CG):
            pltpu.async_copy(
                val_v.at[pl.ds(j * GSZ, GSZ)],
                table.at[idx_v[pb].at[half * CG + j]],
                sout,
                add=True,
            )

    start_idx(0, 0)
    start_vt(0, 0)

    def qbody(q, carry):
        for k in range(4):
            cc = 4 * q + k
            b = k & 1
            p = 2 * q + (k >> 1)
            pb = (k >> 1) & 1

            @pl.when(cc < my_n)
            def _(k=k, cc=cc, b=b, p=p, pb=pb):
                if k % 2 == 0:
                    drain_idx(p, pb)

                if k % 2 == 1:
                    @pl.when(cc + 1 < my_n)
                    def _():
                        start_idx(p + 1, 1 - pb)

                drain_vt(cc, b)

                @pl.when(cc + 1 < my_n)
                def _():
                    start_vt(cc + 1, 1 - b)

                @pl.when(cc >= 1)
                def _():
                    drain_out()

                compute(b)
                scatter(pb, k & 1)
        return carry

    lax.fori_loop(0, QMAX, qbody, 0)

    drain_out()

    plsc.subcore_barrier()

    def dump(rbase, nrows):
        pltpu.sync_copy(
            table.at[pl.ds(rbase, nrows)], val_v.at[pl.ds(0, nrows)]
        )
        pltpu.sync_copy(
            val_v.at[pl.ds(0, nrows)], part_hbm.at[cid, pl.ds(rbase, nrows)]
        )

    for t in range(zfull):
        dump(r0 + t * CE, CE)
    dump(r0 + zfull * CE, zrem)

    @pl.when(sid < ZEXTRA)
    def _():
        dump(r0 + zrows, 8)


_scatter = pl.kernel(
    _scatter_body,
    out_type=jax.ShapeDtypeStruct((NC, N, PW), jnp.float32),
    mesh=_mesh,
    compiler_params=_params,
    scratch_types=[
        pltpu.VMEM((2 * CG, GSZ), jnp.int32),
        pltpu.VMEM((2 * CG, GSZ), jnp.int32),
        pltpu.VMEM((PW, CE), jnp.float32),
        pltpu.VMEM((PW, CE), jnp.float32),
        pltpu.VMEM((CE, PW), jnp.float32),
        pltpu.VMEM_SHARED((N, PW), jnp.float32),
        pltpu.SemaphoreType.DMA,
        pltpu.SemaphoreType.DMA,
        pltpu.SemaphoreType.DMA,
        pltpu.SemaphoreType.DMA,
        pltpu.SemaphoreType.DMA,
    ],
)


def _finish_body(part_hbm, out_hbm, p0_v, p1_v, o_v):
    cid = lax.axis_index("c")
    sid = lax.axis_index("s")
    wid = sid * NC + cid
    iota = lax.iota(jnp.int32, 16)
    cols = [jnp.full((16,), j, jnp.int32) for j in range(C + 1)]

    gr0 = BASE_R * wid + jnp.minimum(wid, EXTRA_R)

    def process(grb, ng):
        rb = grb * 16
        nr = ng * 16
        pltpu.sync_copy(part_hbm.at[0, pl.ds(rb, nr)], p0_v.at[pl.ds(0, nr)])
        pltpu.sync_copy(part_hbm.at[1, pl.ds(rb, nr)], p1_v.at[pl.ds(0, nr)])

        def gbody(g, row_idx):
            den = jnp.maximum(
                plsc.load_gather(p0_v, [row_idx, cols[C]])
                + plsc.load_gather(p1_v, [row_idx, cols[C]]),
                1e-12,
            )
            for j in range(C):
                s = plsc.load_gather(p0_v, [row_idx, cols[j]]) + plsc.load_gather(
                    p1_v, [row_idx, cols[j]]
                )
                plsc.store_scatter(o_v, [row_idx, cols[j]], s / den)
            return row_idx + 16

        lax.fori_loop(0, ng, gbody, iota)
        pltpu.sync_copy(o_v.at[pl.ds(0, nr)], out_hbm.at[pl.ds(rb, nr)])

    def chunk(cc, carry):
        process(gr0 + cc * CGR, CGR)
        return carry

    lax.fori_loop(0, FULL2, chunk, 0)
    process(gr0 + FULL2 * CGR, TAIL2)

    @pl.when(wid < EXTRA_R)
    def _():
        process(gr0 + BASE_R, 1)


_finish = pl.kernel(
    _finish_body,
    out_type=jax.ShapeDtypeStruct((N, PW), jnp.float32),
    mesh=_mesh,
    compiler_params=_params,
    scratch_types=[
        pltpu.VMEM((CGR * 16, PW), jnp.float32),
        pltpu.VMEM((CGR * 16, PW), jnp.float32),
        pltpu.VMEM((CGR * 16, PW), jnp.float32),
    ],
)


def kernel(poss_edge, weights, edges):
    valt = jnp.concatenate(
        [
            poss_edge.T * weights[None, :],
            weights[None, :],
            jnp.zeros((PW - C - 1, E), jnp.float32),
        ],
        axis=0,
    )
    src2d = edges[:, 0].reshape(NG, GSZ)
    valt, src2d = lax.optimization_barrier((valt, src2d))
    part = _scatter(valt, src2d)
    out = _finish(part)
    return out[:, :C], poss_edge
